# baseline (speedup 1.0000x reference)
import functools

import jax
import jax.numpy as jnp
from jax import lax
from jax.experimental import pallas as pl
from jax.experimental.pallas import tpu as pltpu
from jax.experimental.pallas import tpu_sc as plsc

_D = 60
_DP = 64
_W = 20
_EPS = 1e-12
_NC = 2
_NS = 16
_NW = _NC * _NS
_C = 128
_EV = 16
_LW = 128


def _prescale_body(x_ref, o_ref, *, scale):
    x = x_ref[...]
    n = jnp.sqrt(jnp.sum(x * x, axis=1, keepdims=True))
    y = x * (scale / jnp.maximum(n, _EPS))
    o_ref[...] = jnp.concatenate(
        [y, jnp.zeros((y.shape[0], _DP - _D), y.dtype)], axis=1)


def _prescale(table, scale, blk):
    n = table.shape[0]
    return pl.pallas_call(
        functools.partial(_prescale_body, scale=scale),
        grid=(n // blk,),
        in_specs=[pl.BlockSpec((blk, _D), lambda i: (i, 0))],
        out_specs=pl.BlockSpec((blk, _DP), lambda i: (i, 0)),
        out_shape=jax.ShapeDtypeStruct((n, _DP), jnp.float32),
    )(table)


def _sc_words(B, idxs, words_t, zeros_c, word_s, ebias16, rbias16):
    nb = B // _NW
    nchunks = nb // _C
    f32 = jnp.float32
    mesh = plsc.VectorSubcoreMesh(core_axis_name="c", subcore_axis_name="s")
    out_type = (
        jax.ShapeDtypeStruct((B, _DP), f32),
        jax.ShapeDtypeStruct((B, _DP), f32),
        jax.ShapeDtypeStruct((B, _DP), f32),
        jax.ShapeDtypeStruct((B, _EV), f32),
        jax.ShapeDtypeStruct((B, _EV), f32),
        jax.ShapeDtypeStruct((B, _EV), f32),
    )
    scratch = [
        pltpu.VMEM((3, _C), jnp.int32),
        pltpu.VMEM((3 * _W, _C), jnp.int32),
        pltpu.VMEM((_C, _DP), f32),
        pltpu.VMEM((_C, _DP), f32),
        pltpu.VMEM((_C, _DP), f32),
        pltpu.VMEM((_C, _EV), f32),
        pltpu.VMEM((_C, _EV), f32),
        pltpu.VMEM((_C, _EV), f32),
        pltpu.SemaphoreType.DMA,
    ]

    @functools.partial(
        pl.kernel, out_type=out_type, mesh=mesh, scratch_types=scratch,
        compiler_params=pltpu.CompilerParams(use_tc_tiling_on_sc=False))
    def k(idxs_r, wt_r, z_r, wds_r, eb_r, rb_r,
          o_ah, o_ar, o_at, o_hb, o_tb, o_rb,
          sidx, widx, bah, bar, bat, bhb, btb, brb, sem):
        wid = lax.axis_index("s") * _NC + lax.axis_index("c")
        base0 = wid * nb

        def chunk(ci, carry):
            base = base0 + ci * _C
            pltpu.sync_copy(idxs_r.at[:, pl.ds(base, _C)], sidx)
            pltpu.sync_copy(wt_r.at[:, pl.ds(base, _C)], widx)
            pltpu.sync_copy(z_r, bah)
            pltpu.sync_copy(z_r, bar)
            pltpu.sync_copy(z_r, bat)
            cps = []
            for s, accbuf in enumerate((bah, bar, bat)):
                for w in range(_W):
                    cps.append(pltpu.async_copy(
                        wds_r.at[widx.at[s * _W + w]], accbuf, sem,
                        add=True))
            cps.append(pltpu.async_copy(eb_r.at[sidx.at[0]], bhb, sem))
            cps.append(pltpu.async_copy(eb_r.at[sidx.at[1]], btb, sem))
            cps.append(pltpu.async_copy(rb_r.at[sidx.at[2]], brb, sem))
            for cp in cps:
                cp.wait()
            pltpu.sync_copy(bah, o_ah.at[pl.ds(base, _C)])
            pltpu.sync_copy(bar, o_ar.at[pl.ds(base, _C)])
            pltpu.sync_copy(bat, o_at.at[pl.ds(base, _C)])
            pltpu.sync_copy(bhb, o_hb.at[pl.ds(base, _C)])
            pltpu.sync_copy(btb, o_tb.at[pl.ds(base, _C)])
            pltpu.sync_copy(brb, o_rb.at[pl.ds(base, _C)])
            return carry

        lax.fori_loop(0, nchunks, chunk, 0)

    return k(idxs, words_t, zeros_c, word_s, ebias16, rbias16)


def _entpad_body(x_ref, o_ref):
    x = x_ref[...]
    n = jnp.sqrt(jnp.sum(x * x, axis=1, keepdims=True))
    y = x / jnp.maximum(n, _EPS)
    o_ref[...] = jnp.concatenate(
        [y, jnp.zeros((y.shape[0], _LW - _D), y.dtype)], axis=1)


def _entpad(table, blk):
    n = table.shape[0]
    return pl.pallas_call(
        _entpad_body,
        grid=(n // blk,),
        in_specs=[pl.BlockSpec((blk, _D), lambda i: (i, 0))],
        out_specs=pl.BlockSpec((blk, _LW), lambda i: (i, 0)),
        out_shape=jax.ShapeDtypeStruct((n, _LW), jnp.float32),
    )(table)


_GS = 16


def _sc_entity(B, idxs, ent, rel_t, proj_t):
    nb = B // _NW
    nchunks = nb // _C
    f32 = jnp.float32
    mesh = plsc.VectorSubcoreMesh(core_axis_name="c", subcore_axis_name="s")
    out_type = (
        jax.ShapeDtypeStruct((B, _D), f32),
        jax.ShapeDtypeStruct((B, _D), f32),
        jax.ShapeDtypeStruct((B, _D), f32),
        jax.ShapeDtypeStruct((B, _D), f32),
    )
    scratch = [
        pltpu.VMEM((3, _C), jnp.int32),
        pltpu.VMEM((_C, _D), f32),
        pltpu.VMEM((_C, _D), f32),
        pltpu.VMEM((_C, _D), f32),
        pltpu.VMEM((_C, _D), f32),
        pltpu.SemaphoreType.DMA,
    ]

    @functools.partial(pl.kernel, out_type=out_type, mesh=mesh,
                       scratch_types=scratch)
    def k(idxs_r, ent_r, rl_r, pj_r, o_eh, o_et, o_rl, o_pj,
          sidx, beh, bet, brl, bpj, sem):
        wid = lax.axis_index("s") * _NC + lax.axis_index("c")
        base0 = wid * nb

        def chunk(ci, carry):
            base = base0 + ci * _C
            pltpu.sync_copy(idxs_r.at[:, pl.ds(base, _C)], sidx)

            def group(gi, carry2):
                g0 = gi * _GS
                hvec = sidx[0, pl.ds(g0, _GS)]
                tvec = sidx[1, pl.ds(g0, _GS)]
                rvec = sidx[2, pl.ds(g0, _GS)]
                for i in range(_GS):
                    pltpu.async_copy(
                        ent_r.at[pl.ds(hvec[i], 1)],
                        beh.at[pl.ds(g0 + i, 1)], sem)
                    pltpu.async_copy(
                        ent_r.at[pl.ds(tvec[i], 1)],
                        bet.at[pl.ds(g0 + i, 1)], sem)
                    pltpu.async_copy(
                        rl_r.at[pl.ds(rvec[i], 1)],
                        brl.at[pl.ds(g0 + i, 1)], sem)
                    pltpu.async_copy(
                        pj_r.at[pl.ds(rvec[i], 1)],
                        bpj.at[pl.ds(g0 + i, 1)], sem)
                for buf in (beh, bet, brl, bpj):
                    pltpu.make_async_copy(
                        ent_r.at[pl.ds(0, _GS)], buf.at[pl.ds(g0, _GS)],
                        sem).wait()
                return carry2

            lax.fori_loop(0, _C // _GS, group, 0)
            pltpu.sync_copy(beh, o_eh.at[pl.ds(base, _C)])
            pltpu.sync_copy(bet, o_et.at[pl.ds(base, _C)])
            pltpu.sync_copy(brl, o_rl.at[pl.ds(base, _C)])
            pltpu.sync_copy(bpj, o_pj.at[pl.ds(base, _C)])
            return carry

        lax.fori_loop(0, nchunks, chunk, 0)

    return k(idxs, ent, rel_t, proj_t)


def _score_body(eh, et, rl, pj, ah, ar, at_, hb, tb, rb, aux, o):
    a = aux[...]
    blk = a.shape[0]
    iota16 = lax.broadcasted_iota(jnp.int32, (blk, _EV), 1)

    def lane_pick(x, lane):
        return jnp.sum(jnp.where(iota16 == lane, x, 0.0), axis=1)

    def nrm(x):
        n = jnp.sqrt(jnp.sum(x * x, axis=1, keepdims=True))
        return x / jnp.maximum(n, _EPS)

    head_e = nrm(eh[...]) + ah[...][:, :_D]
    tail_e = nrm(et[...]) + at_[...][:, :_D]
    rel_e = nrm(rl[...]) + ar[...][:, :_D]
    p = nrm(pj[...])
    hp = head_e - jnp.sum(p * head_e, axis=1, keepdims=True) * p
    tp = tail_e - jnp.sum(p * tail_e, axis=1, keepdims=True) * p
    diff = hp + rel_e - tp
    sc = -jnp.sqrt(jnp.sum(diff * diff, axis=1))
    o[...] = (sc + lane_pick(hb[...], a[:, 0:1])
              + lane_pick(tb[...], a[:, 1:2])
              + lane_pick(rb[...], a[:, 2:3]))


def _score(B, eh, et, rl, pj, ah, ar, at_, hb, tb, rb, aux):
    blk = 2048
    specE = pl.BlockSpec((blk, _D), lambda i: (i, 0))
    specP = pl.BlockSpec((blk, _DP), lambda i: (i, 0))
    specV = pl.BlockSpec((blk, _EV), lambda i: (i, 0))
    specA = pl.BlockSpec((blk, 8), lambda i: (i, 0))
    return pl.pallas_call(
        _score_body,
        grid=(B // blk,),
        in_specs=[specE, specE, specE, specE, specP, specP, specP,
                  specV, specV, specV, specA],
        out_specs=pl.BlockSpec((blk,), lambda i: (i,)),
        out_shape=jax.ShapeDtypeStruct((B,), jnp.float32),
    )(eh, et, rl, pj, ah, ar, at_, hb, tb, rb, aux)


def kernel(head, relation, tail, head_w, rel_w, tail_w, entity_embedding,
           relation_embedding, word_embedding, e_bias, r_bias,
           relation_projection):
    B = head.shape[0]
    i32 = jnp.int32
    head = head.astype(i32)
    tail = tail.astype(i32)
    relation = relation.astype(i32)
    words_t = jnp.concatenate(
        [head_w.T.astype(i32), rel_w.T.astype(i32), tail_w.T.astype(i32)],
        axis=0)
    idxs_e = jnp.stack([head, tail, relation], axis=0)
    idxs_w = jnp.stack([head >> 4, tail >> 4, relation >> 4], axis=0)
    aux = jnp.stack([
        head & (_EV - 1), tail & (_EV - 1), relation & (_EV - 1),
        jnp.zeros_like(head), jnp.zeros_like(head), jnp.zeros_like(head),
        jnp.zeros_like(head), jnp.zeros_like(head),
    ], axis=1)

    word_s = _prescale(word_embedding, 1.0 / _W, 2000)
    zeros_c = jnp.zeros((_C, _DP), jnp.float32)

    ebias16 = e_bias.reshape(-1, _EV)
    nr = r_bias.shape[0]
    rpad = (-nr) % _EV
    rbias16 = jnp.concatenate(
        [r_bias, jnp.zeros((rpad, 1), r_bias.dtype)], axis=0).reshape(-1, _EV)

    ah, ar, at_, hb, tb, rb = _sc_words(
        B, idxs_w, words_t, zeros_c, word_s, ebias16, rbias16)
    eh, et, rl, pj = _sc_entity(B, idxs_e, entity_embedding,
                                relation_embedding, relation_projection)
    return _score(B, eh, et, rl, pj, ah, ar, at_, hb, tb, rb, aux)


# speedup vs baseline: 1.0242x; 1.0242x over previous
import functools

import jax
import jax.numpy as jnp
from jax import lax
from jax.experimental import pallas as pl
from jax.experimental.pallas import tpu as pltpu
from jax.experimental.pallas import tpu_sc as plsc

_D = 60
_DP = 64
_W = 20
_EPS = 1e-12
_NC = 2
_NS = 16
_NW = _NC * _NS
_C = 128
_EV = 16
_LW = 128


def _prescale_body(x_ref, o_ref, *, scale):
    x = x_ref[...]
    n = jnp.sqrt(jnp.sum(x * x, axis=1, keepdims=True))
    y = (x * (scale / jnp.maximum(n, _EPS))).astype(o_ref.dtype)
    o_ref[...] = jnp.concatenate(
        [y, jnp.zeros((y.shape[0], _DP - _D), y.dtype)], axis=1)


def _prescale(table, scale, blk, dtype):
    n = table.shape[0]
    return pl.pallas_call(
        functools.partial(_prescale_body, scale=scale),
        grid=(n // blk,),
        in_specs=[pl.BlockSpec((blk, _D), lambda i: (i, 0))],
        out_specs=pl.BlockSpec((blk, _DP), lambda i: (i, 0)),
        out_shape=jax.ShapeDtypeStruct((n, _DP), dtype),
    )(table)


def _sc_words(B, idxs, words_t, zeros_c, word_s, ebias16, rbias16):
    nb = B // _NW
    nchunks = nb // _C
    f32 = jnp.float32
    mesh = plsc.VectorSubcoreMesh(core_axis_name="c", subcore_axis_name="s")
    bf16 = jnp.bfloat16
    out_type = (
        jax.ShapeDtypeStruct((B, _DP), bf16),
        jax.ShapeDtypeStruct((B, _DP), bf16),
        jax.ShapeDtypeStruct((B, _DP), bf16),
        jax.ShapeDtypeStruct((B, _EV), f32),
        jax.ShapeDtypeStruct((B, _EV), f32),
        jax.ShapeDtypeStruct((B, _EV), f32),
    )
    scratch = [
        pltpu.VMEM((3, _C), jnp.int32),
        pltpu.VMEM((3 * _W, _C), jnp.int32),
        pltpu.VMEM((_C, _DP), bf16),
        pltpu.VMEM((_C, _DP), bf16),
        pltpu.VMEM((_C, _DP), bf16),
        pltpu.VMEM((_C, _EV), f32),
        pltpu.VMEM((_C, _EV), f32),
        pltpu.VMEM((_C, _EV), f32),
        pltpu.SemaphoreType.DMA,
    ]

    @functools.partial(
        pl.kernel, out_type=out_type, mesh=mesh, scratch_types=scratch,
        compiler_params=pltpu.CompilerParams(use_tc_tiling_on_sc=False))
    def k(idxs_r, wt_r, z_r, wds_r, eb_r, rb_r,
          o_ah, o_ar, o_at, o_hb, o_tb, o_rb,
          sidx, widx, bah, bar, bat, bhb, btb, brb, sem):
        wid = lax.axis_index("s") * _NC + lax.axis_index("c")
        base0 = wid * nb

        def chunk(ci, carry):
            base = base0 + ci * _C
            pltpu.sync_copy(idxs_r.at[:, pl.ds(base, _C)], sidx)
            pltpu.sync_copy(wt_r.at[:, pl.ds(base, _C)], widx)
            pltpu.sync_copy(z_r, bah)
            pltpu.sync_copy(z_r, bar)
            pltpu.sync_copy(z_r, bat)
            cps = []
            for s, accbuf in enumerate((bah, bar, bat)):
                for w in range(_W):
                    cps.append(pltpu.async_copy(
                        wds_r.at[widx.at[s * _W + w]], accbuf, sem,
                        add=True))
            cps.append(pltpu.async_copy(eb_r.at[sidx.at[0]], bhb, sem))
            cps.append(pltpu.async_copy(eb_r.at[sidx.at[1]], btb, sem))
            cps.append(pltpu.async_copy(rb_r.at[sidx.at[2]], brb, sem))
            for cp in cps:
                cp.wait()
            pltpu.sync_copy(bah, o_ah.at[pl.ds(base, _C)])
            pltpu.sync_copy(bar, o_ar.at[pl.ds(base, _C)])
            pltpu.sync_copy(bat, o_at.at[pl.ds(base, _C)])
            pltpu.sync_copy(bhb, o_hb.at[pl.ds(base, _C)])
            pltpu.sync_copy(btb, o_tb.at[pl.ds(base, _C)])
            pltpu.sync_copy(brb, o_rb.at[pl.ds(base, _C)])
            return carry

        lax.fori_loop(0, nchunks, chunk, 0)

    return k(idxs, words_t, zeros_c, word_s, ebias16, rbias16)


def _entpad_body(x_ref, o_ref):
    x = x_ref[...]
    n = jnp.sqrt(jnp.sum(x * x, axis=1, keepdims=True))
    y = x / jnp.maximum(n, _EPS)
    o_ref[...] = jnp.concatenate(
        [y, jnp.zeros((y.shape[0], _LW - _D), y.dtype)], axis=1)


def _entpad(table, blk):
    n = table.shape[0]
    return pl.pallas_call(
        _entpad_body,
        grid=(n // blk,),
        in_specs=[pl.BlockSpec((blk, _D), lambda i: (i, 0))],
        out_specs=pl.BlockSpec((blk, _LW), lambda i: (i, 0)),
        out_shape=jax.ShapeDtypeStruct((n, _LW), jnp.float32),
    )(table)


_GS = 16


def _sc_entity(B, idxs, ent, rel_t, proj_t):
    nb = B // _NW
    nchunks = nb // _C
    f32 = jnp.float32
    mesh = plsc.VectorSubcoreMesh(core_axis_name="c", subcore_axis_name="s")
    out_type = (
        jax.ShapeDtypeStruct((B, _D), f32),
        jax.ShapeDtypeStruct((B, _D), f32),
        jax.ShapeDtypeStruct((B, _D), f32),
        jax.ShapeDtypeStruct((B, _D), f32),
    )
    scratch = [
        pltpu.VMEM((3, _C), jnp.int32),
        pltpu.VMEM((_C, _D), f32),
        pltpu.VMEM((_C, _D), f32),
        pltpu.VMEM((_C, _D), f32),
        pltpu.VMEM((_C, _D), f32),
        pltpu.SemaphoreType.DMA,
    ]

    @functools.partial(pl.kernel, out_type=out_type, mesh=mesh,
                       scratch_types=scratch)
    def k(idxs_r, ent_r, rl_r, pj_r, o_eh, o_et, o_rl, o_pj,
          sidx, beh, bet, brl, bpj, sem):
        wid = lax.axis_index("s") * _NC + lax.axis_index("c")
        base0 = wid * nb

        def chunk(ci, carry):
            base = base0 + ci * _C
            pltpu.sync_copy(idxs_r.at[:, pl.ds(base, _C)], sidx)

            def group(gi, carry2):
                g0 = gi * _GS
                hvec = sidx[0, pl.ds(g0, _GS)]
                tvec = sidx[1, pl.ds(g0, _GS)]
                rvec = sidx[2, pl.ds(g0, _GS)]
                for i in range(_GS):
                    pltpu.async_copy(
                        ent_r.at[pl.ds(hvec[i], 1)],
                        beh.at[pl.ds(g0 + i, 1)], sem)
                    pltpu.async_copy(
                        ent_r.at[pl.ds(tvec[i], 1)],
                        bet.at[pl.ds(g0 + i, 1)], sem)
                    pltpu.async_copy(
                        rl_r.at[pl.ds(rvec[i], 1)],
                        brl.at[pl.ds(g0 + i, 1)], sem)
                    pltpu.async_copy(
                        pj_r.at[pl.ds(rvec[i], 1)],
                        bpj.at[pl.ds(g0 + i, 1)], sem)
                for buf in (beh, bet, brl, bpj):
                    pltpu.make_async_copy(
                        ent_r.at[pl.ds(0, _GS)], buf.at[pl.ds(g0, _GS)],
                        sem).wait()
                return carry2

            lax.fori_loop(0, _C // _GS, group, 0)
            pltpu.sync_copy(beh, o_eh.at[pl.ds(base, _C)])
            pltpu.sync_copy(bet, o_et.at[pl.ds(base, _C)])
            pltpu.sync_copy(brl, o_rl.at[pl.ds(base, _C)])
            pltpu.sync_copy(bpj, o_pj.at[pl.ds(base, _C)])
            return carry

        lax.fori_loop(0, nchunks, chunk, 0)

    return k(idxs, ent, rel_t, proj_t)


def _score_body(eh, et, rl, pj, ah, ar, at_, hb, tb, rb, aux, o):
    a = aux[...]
    blk = a.shape[0]
    iota16 = lax.broadcasted_iota(jnp.int32, (blk, _EV), 1)

    def lane_pick(x, lane):
        return jnp.sum(jnp.where(iota16 == lane, x, 0.0), axis=1)

    def nrm(x):
        n = jnp.sqrt(jnp.sum(x * x, axis=1, keepdims=True))
        return x / jnp.maximum(n, _EPS)

    f32 = jnp.float32
    head_e = nrm(eh[...]) + ah[...][:, :_D].astype(f32)
    tail_e = nrm(et[...]) + at_[...][:, :_D].astype(f32)
    rel_e = nrm(rl[...]) + ar[...][:, :_D].astype(f32)
    p = nrm(pj[...])
    hp = head_e - jnp.sum(p * head_e, axis=1, keepdims=True) * p
    tp = tail_e - jnp.sum(p * tail_e, axis=1, keepdims=True) * p
    diff = hp + rel_e - tp
    sc = -jnp.sqrt(jnp.sum(diff * diff, axis=1))
    o[...] = (sc + lane_pick(hb[...], a[:, 0:1])
              + lane_pick(tb[...], a[:, 1:2])
              + lane_pick(rb[...], a[:, 2:3]))


def _score(B, eh, et, rl, pj, ah, ar, at_, hb, tb, rb, aux):
    blk = 2048
    specE = pl.BlockSpec((blk, _D), lambda i: (i, 0))
    specP = pl.BlockSpec((blk, _DP), lambda i: (i, 0))
    specV = pl.BlockSpec((blk, _EV), lambda i: (i, 0))
    specA = pl.BlockSpec((blk, 8), lambda i: (i, 0))
    return pl.pallas_call(
        _score_body,
        grid=(B // blk,),
        in_specs=[specE, specE, specE, specE, specP, specP, specP,
                  specV, specV, specV, specA],
        out_specs=pl.BlockSpec((blk,), lambda i: (i,)),
        out_shape=jax.ShapeDtypeStruct((B,), jnp.float32),
    )(eh, et, rl, pj, ah, ar, at_, hb, tb, rb, aux)


def kernel(head, relation, tail, head_w, rel_w, tail_w, entity_embedding,
           relation_embedding, word_embedding, e_bias, r_bias,
           relation_projection):
    B = head.shape[0]
    i32 = jnp.int32
    head = head.astype(i32)
    tail = tail.astype(i32)
    relation = relation.astype(i32)
    words_t = jnp.concatenate(
        [head_w.T.astype(i32), rel_w.T.astype(i32), tail_w.T.astype(i32)],
        axis=0)
    idxs_e = jnp.stack([head, tail, relation], axis=0)
    idxs_w = jnp.stack([head >> 4, tail >> 4, relation >> 4], axis=0)
    aux = jnp.stack([
        head & (_EV - 1), tail & (_EV - 1), relation & (_EV - 1),
        jnp.zeros_like(head), jnp.zeros_like(head), jnp.zeros_like(head),
        jnp.zeros_like(head), jnp.zeros_like(head),
    ], axis=1)

    word_s = _prescale(word_embedding, 1.0 / _W, 2000, jnp.bfloat16)
    zeros_c = jnp.zeros((_C, _DP), jnp.bfloat16)

    ebias16 = e_bias.reshape(-1, _EV)
    nr = r_bias.shape[0]
    rpad = (-nr) % _EV
    rbias16 = jnp.concatenate(
        [r_bias, jnp.zeros((rpad, 1), r_bias.dtype)], axis=0).reshape(-1, _EV)

    ah, ar, at_, hb, tb, rb = _sc_words(
        B, idxs_w, words_t, zeros_c, word_s, ebias16, rbias16)
    eh, et, rl, pj = _sc_entity(B, idxs_e, entity_embedding,
                                relation_embedding, relation_projection)
    return _score(B, eh, et, rl, pj, ah, ar, at_, hb, tb, rb, aux)
